# baseline (speedup 1.0000x reference)
import functools

import jax
import jax.numpy as jnp
from jax import lax
from jax.experimental import pallas as pl
from jax.experimental.pallas import tpu as pltpu
from jax.experimental.pallas import tpu_sc as plsc

_NUM_FIELDS = 26
_VOCAB = 100000
_EMB_DIM = 16
_BATCH = 16384

try:
    _info = plsc.get_sparse_core_info()
    _NC, _NS, _L = _info.num_cores, _info.num_subcores, _info.num_lanes
except Exception:
    _NC, _NS, _L = 2, 16, 16

_NW = _NC * _NS
_BPW = _BATCH // _NW


def _make_sc_gather():
    mesh = plsc.VectorSubcoreMesh(core_axis_name="c", subcore_axis_name="s")

    @functools.partial(
        pl.kernel,
        out_type=jax.ShapeDtypeStruct(
            (_BATCH, _NUM_FIELDS * _EMB_DIM), jnp.float32),
        mesh=mesh,
        compiler_params=pltpu.CompilerParams(use_tc_tiling_on_sc=False),
        scratch_types=(
            [pltpu.VMEM((_BPW,), jnp.int32) for _ in range(_NUM_FIELDS)]
            + [
                pltpu.VMEM((_BPW, _EMB_DIM), jnp.float32),
                pltpu.VMEM((_BPW, _EMB_DIM), jnp.float32),
                pltpu.SemaphoreType.DMA,
                pltpu.SemaphoreType.DMA,
                pltpu.SemaphoreType.DMA,
            ]
        ),
    )
    def k(tab_hbm, idx_hbm, out_hbm, *scratch):
        idx_vs = scratch[:_NUM_FIELDS]
        buf0, buf1, sem_idx, sem0, sem1 = scratch[_NUM_FIELDS:]
        wid = lax.axis_index("s") * _NC + lax.axis_index("c")
        b0 = wid * _BPW

        idx_descs = [
            pltpu.async_copy(
                idx_hbm.at[f, pl.ds(b0, _BPW)], idx_vs[f], sem_idx)
            for f in range(_NUM_FIELDS)
        ]

        bufs = (buf0, buf1)
        sems = (sem0, sem1)

        def fire(f):
            idx_descs[f].wait()
            return pltpu.async_copy(
                tab_hbm.at[f].at[idx_vs[f]], bufs[f % 2], sems[f % 2])

        def store(f):
            pltpu.sync_copy(
                bufs[f % 2],
                out_hbm.at[pl.ds(b0, _BPW), pl.ds(f * _EMB_DIM, _EMB_DIM)])

        descs = [fire(0)]
        for f in range(1, _NUM_FIELDS):
            descs.append(fire(f))
            descs[f - 1].wait()
            store(f - 1)
        descs[_NUM_FIELDS - 1].wait()
        store(_NUM_FIELDS - 1)

    return k


_sc_gather = _make_sc_gather()


def kernel(fixed_tensor, tables):
    idx_t = fixed_tensor.astype(jnp.int32).T
    return _sc_gather(tables, idx_t)
